# initial kernel scaffold (unmeasured)
import jax
import jax.numpy as jnp
from jax import lax
from jax.experimental import pallas as pl
from jax.experimental.pallas import tpu as pltpu

N_DEV = 4
N_LAYERS = 3
N_HOPS = N_DEV - 1


def kernel(x, Win0, Wout0, Win1, Wout1, Win2, Wout2):
    b, d = x.shape

    def body(
        x_ref,
        win0_ref,
        wout0_ref,
        win1_ref,
        wout1_ref,
        win2_ref,
        wout2_ref,
        out_ref,
        comm_ref,
        send_sems,
        recv_sems,
    ):
        my_pos = lax.axis_index("i")
        left = (my_pos + N_DEV - 1) % N_DEV
        right = (my_pos + 1) % N_DEV

        barrier_sem = pltpu.get_barrier_semaphore()
        for nbr in (left, right):
            pl.semaphore_signal(
                barrier_sem,
                inc=1,
                device_id=(nbr,),
                device_id_type=pl.DeviceIdType.MESH,
            )
        pl.semaphore_wait(barrier_sem, 2)

        wins = [win0_ref, win1_ref, win2_ref]
        wouts = [wout0_ref, wout1_ref, wout2_ref]

        xv = x_ref[:, :]
        for layer in range(N_LAYERS):
            h = jnp.maximum(
                jnp.dot(xv, wins[layer][:, :], preferred_element_type=jnp.float32),
                0.0,
            )
            partial = jnp.dot(
                h, wouts[layer][:, :], preferred_element_type=jnp.float32
            )

            comm_ref[0] = partial
            acc = partial
            for hop in range(N_HOPS):
                s = layer * N_HOPS + hop
                rdma = pltpu.make_async_remote_copy(
                    src_ref=comm_ref.at[hop],
                    dst_ref=comm_ref.at[hop + 1],
                    send_sem=send_sems.at[s],
                    recv_sem=recv_sems.at[s],
                    device_id=(right,),
                    device_id_type=pl.DeviceIdType.MESH,
                )
                rdma.start()
                rdma.wait()
                acc = acc + comm_ref[hop + 1]
            xv = acc

        out_ref[:, :] = xv

    return pl.pallas_call(
        body,
        out_shape=jax.ShapeDtypeStruct((b, d), jnp.float32),
        in_specs=[pl.BlockSpec(memory_space=pltpu.VMEM)] * 7,
        out_specs=pl.BlockSpec(memory_space=pltpu.VMEM),
        scratch_shapes=[
            pltpu.VMEM((N_HOPS + 1, b, d), jnp.float32),
            pltpu.SemaphoreType.DMA((N_LAYERS * N_HOPS,)),
            pltpu.SemaphoreType.DMA((N_LAYERS * N_HOPS,)),
        ],
        compiler_params=pltpu.CompilerParams(collective_id=0),
    )(x, Win0, Wout0, Win1, Wout1, Win2, Wout2)


# baseline (device time: 66967 ns/iter reference)
import jax
import jax.numpy as jnp
from jax import lax
from jax.experimental import pallas as pl
from jax.experimental.pallas import tpu as pltpu

N_DEV = 4
N_LAYERS = 3
N_HOPS = N_DEV - 1


def kernel(x, Win0, Wout0, Win1, Wout1, Win2, Wout2):
    b, d = x.shape

    def body(
        x_ref,
        win0_ref,
        wout0_ref,
        win1_ref,
        wout1_ref,
        win2_ref,
        wout2_ref,
        out_ref,
        comm_ref,
        send_sems,
        recv_sems,
    ):
        my_pos = lax.axis_index("i")
        left = (my_pos + N_DEV - 1) % N_DEV
        right = (my_pos + 1) % N_DEV

        barrier_sem = pltpu.get_barrier_semaphore()
        for nbr in (left, right):
            pl.semaphore_signal(
                barrier_sem,
                inc=1,
                device_id=(nbr,),
                device_id_type=pl.DeviceIdType.MESH,
            )
        pl.semaphore_wait(barrier_sem, 2)

        wins = [win0_ref, win1_ref, win2_ref]
        wouts = [wout0_ref, wout1_ref, wout2_ref]

        xv = x_ref[:, :]
        for layer in range(N_LAYERS):
            h = jnp.maximum(
                jnp.dot(xv, wins[layer][:, :], preferred_element_type=jnp.float32),
                0.0,
            )
            partial = jnp.dot(
                h, wouts[layer][:, :], preferred_element_type=jnp.float32
            )

            comm_ref[0] = partial
            acc = partial
            for hop in range(N_HOPS):
                s = layer * N_HOPS + hop
                rdma = pltpu.make_async_remote_copy(
                    src_ref=comm_ref.at[hop],
                    dst_ref=comm_ref.at[hop + 1],
                    send_sem=send_sems.at[s],
                    recv_sem=recv_sems.at[s],
                    device_id=(right,),
                    device_id_type=pl.DeviceIdType.MESH,
                )
                rdma.start()
                rdma.wait()
                acc = acc + comm_ref[hop + 1]
            xv = acc

        out_ref[:, :] = xv

    return pl.pallas_call(
        body,
        out_shape=jax.ShapeDtypeStruct((b, d), jnp.float32),
        in_specs=[pl.BlockSpec(memory_space=pltpu.VMEM)] * 7,
        out_specs=pl.BlockSpec(memory_space=pltpu.VMEM),
        scratch_shapes=[
            pltpu.VMEM((N_HOPS + 1, b, d), jnp.float32),
            pltpu.SemaphoreType.DMA((N_LAYERS * N_HOPS,)),
            pltpu.SemaphoreType.DMA((N_LAYERS * N_HOPS,)),
        ],
        compiler_params=pltpu.CompilerParams(
            collective_id=0,
            vmem_limit_bytes=100 * 1024 * 1024,
        ),
    )(x, Win0, Wout0, Win1, Wout1, Win2, Wout2)


# device time: 54351 ns/iter; 1.2321x vs baseline; 1.2321x over previous
import jax
import jax.numpy as jnp
from jax import lax
from jax.experimental import pallas as pl
from jax.experimental.pallas import tpu as pltpu

N_DEV = 4
N_LAYERS = 3
N_STAGES = 2


def kernel(x, Win0, Wout0, Win1, Wout1, Win2, Wout2):
    b, d = x.shape

    def body(
        x_ref,
        win0_ref,
        wout0_ref,
        win1_ref,
        wout1_ref,
        win2_ref,
        wout2_ref,
        out_ref,
        send_ref,
        recv_ref,
        send_sems,
        recv_sems,
    ):
        my_pos = lax.axis_index("i")
        partner = [my_pos ^ 1, 3 - my_pos]

        barrier_sem = pltpu.get_barrier_semaphore()
        for p in partner:
            pl.semaphore_signal(
                barrier_sem,
                inc=1,
                device_id=(p,),
                device_id_type=pl.DeviceIdType.MESH,
            )
        pl.semaphore_wait(barrier_sem, 2)

        wins = [win0_ref, win1_ref, win2_ref]
        wouts = [wout0_ref, wout1_ref, wout2_ref]

        xv = x_ref[:, :]
        for layer in range(N_LAYERS):
            h = jnp.maximum(
                jnp.dot(xv, wins[layer][:, :], preferred_element_type=jnp.float32),
                0.0,
            )
            acc = jnp.dot(h, wouts[layer][:, :], preferred_element_type=jnp.float32)

            for stage in range(N_STAGES):
                s = layer * N_STAGES + stage
                send_ref[stage] = acc
                rdma = pltpu.make_async_remote_copy(
                    src_ref=send_ref.at[stage],
                    dst_ref=recv_ref.at[stage],
                    send_sem=send_sems.at[s],
                    recv_sem=recv_sems.at[s],
                    device_id=(partner[stage],),
                    device_id_type=pl.DeviceIdType.MESH,
                )
                rdma.start()
                rdma.wait()
                acc = acc + recv_ref[stage]
            xv = acc

        out_ref[:, :] = xv

    return pl.pallas_call(
        body,
        out_shape=jax.ShapeDtypeStruct((b, d), jnp.float32),
        in_specs=[pl.BlockSpec(memory_space=pltpu.VMEM)] * 7,
        out_specs=pl.BlockSpec(memory_space=pltpu.VMEM),
        scratch_shapes=[
            pltpu.VMEM((N_STAGES, b, d), jnp.float32),
            pltpu.VMEM((N_STAGES, b, d), jnp.float32),
            pltpu.SemaphoreType.DMA((N_LAYERS * N_STAGES,)),
            pltpu.SemaphoreType.DMA((N_LAYERS * N_STAGES,)),
        ],
        compiler_params=pltpu.CompilerParams(
            collective_id=0,
            vmem_limit_bytes=100 * 1024 * 1024,
        ),
    )(x, Win0, Wout0, Win1, Wout1, Win2, Wout2)


# device time: 48703 ns/iter; 1.3750x vs baseline; 1.1160x over previous
import jax
import jax.numpy as jnp
from jax import lax
from jax.experimental import pallas as pl
from jax.experimental.pallas import tpu as pltpu

N_DEV = 4
N_LAYERS = 3
N_STAGES = 2
N_CHUNKS = 2


def kernel(x, Win0, Wout0, Win1, Wout1, Win2, Wout2):
    b, d = x.shape
    dc = d // N_CHUNKS

    def body(
        x_ref,
        win0_ref,
        wout0_ref,
        win1_ref,
        wout1_ref,
        win2_ref,
        wout2_ref,
        out_ref,
        send_ref,
        recv_ref,
        send_sems,
        recv_sems,
    ):
        my_pos = lax.axis_index("i")
        partner = [my_pos ^ 1, 3 - my_pos]

        barrier_sem = pltpu.get_barrier_semaphore()
        for p in partner:
            pl.semaphore_signal(
                barrier_sem,
                inc=1,
                device_id=(p,),
                device_id_type=pl.DeviceIdType.MESH,
            )
        pl.semaphore_wait(barrier_sem, 2)

        wins = [win0_ref, win1_ref, win2_ref]
        wouts = [wout0_ref, wout1_ref, wout2_ref]

        def exchange(layer, stage, chunk, value):
            send_ref[stage, chunk] = value
            s = (layer * N_STAGES + stage) * N_CHUNKS + chunk
            rdma = pltpu.make_async_remote_copy(
                src_ref=send_ref.at[stage, chunk],
                dst_ref=recv_ref.at[stage, chunk],
                send_sem=send_sems.at[s],
                recv_sem=recv_sems.at[s],
                device_id=(partner[stage],),
                device_id_type=pl.DeviceIdType.MESH,
            )
            rdma.start()
            return rdma

        h = jnp.maximum(
            jnp.dot(x_ref[:, :], win0_ref[:, :], preferred_element_type=jnp.float32),
            0.0,
        )
        for layer in range(N_LAYERS):
            wout = wouts[layer]
            pA = jnp.dot(h, wout[:, :dc], preferred_element_type=jnp.float32)
            rA1 = exchange(layer, 0, 0, pA)
            pB = jnp.dot(h, wout[:, dc:], preferred_element_type=jnp.float32)
            rB1 = exchange(layer, 0, 1, pB)

            rA1.wait()
            accA = pA + recv_ref[0, 0]
            rA2 = exchange(layer, 1, 0, accA)
            rB1.wait()
            accB = pB + recv_ref[0, 1]
            rB2 = exchange(layer, 1, 1, accB)

            rA2.wait()
            xA = accA + recv_ref[1, 0]
            if layer == N_LAYERS - 1:
                out_ref[:, :dc] = xA
                rB2.wait()
                out_ref[:, dc:] = accB + recv_ref[1, 1]
            else:
                win = wins[layer + 1]
                t = jnp.dot(xA, win[:dc, :], preferred_element_type=jnp.float32)
                rB2.wait()
                xB = accB + recv_ref[1, 1]
                h = jnp.maximum(
                    t + jnp.dot(xB, win[dc:, :], preferred_element_type=jnp.float32),
                    0.0,
                )

    return pl.pallas_call(
        body,
        out_shape=jax.ShapeDtypeStruct((b, d), jnp.float32),
        in_specs=[pl.BlockSpec(memory_space=pltpu.VMEM)] * 7,
        out_specs=pl.BlockSpec(memory_space=pltpu.VMEM),
        scratch_shapes=[
            pltpu.VMEM((N_STAGES, N_CHUNKS, b, dc), jnp.float32),
            pltpu.VMEM((N_STAGES, N_CHUNKS, b, dc), jnp.float32),
            pltpu.SemaphoreType.DMA((N_LAYERS * N_STAGES * N_CHUNKS,)),
            pltpu.SemaphoreType.DMA((N_LAYERS * N_STAGES * N_CHUNKS,)),
        ],
        compiler_params=pltpu.CompilerParams(
            collective_id=0,
            vmem_limit_bytes=100 * 1024 * 1024,
        ),
    )(x, Win0, Wout0, Win1, Wout1, Win2, Wout2)


# device time: 44316 ns/iter; 1.5111x vs baseline; 1.0990x over previous
import jax
import jax.numpy as jnp
from jax import lax
from jax.experimental import pallas as pl
from jax.experimental.pallas import tpu as pltpu

N_DEV = 4
N_LAYERS = 3
N_STAGES = 2
N_CHUNKS = 4


def kernel(x, Win0, Wout0, Win1, Wout1, Win2, Wout2):
    b, d = x.shape
    dc = d // N_CHUNKS

    def body(
        x_ref,
        win0_ref,
        wout0_ref,
        win1_ref,
        wout1_ref,
        win2_ref,
        wout2_ref,
        out_ref,
        send_ref,
        recv_ref,
        send_sems,
        recv_sems,
    ):
        my_pos = lax.axis_index("i")
        partner = [my_pos ^ 1, 3 - my_pos]

        barrier_sem = pltpu.get_barrier_semaphore()
        for p in partner:
            pl.semaphore_signal(
                barrier_sem,
                inc=1,
                device_id=(p,),
                device_id_type=pl.DeviceIdType.MESH,
            )
        pl.semaphore_wait(barrier_sem, 2)

        wins = [win0_ref, win1_ref, win2_ref]
        wouts = [wout0_ref, wout1_ref, wout2_ref]

        def exchange(layer, stage, chunk, value):
            send_ref[stage, chunk] = value
            s = (layer * N_STAGES + stage) * N_CHUNKS + chunk
            rdma = pltpu.make_async_remote_copy(
                src_ref=send_ref.at[stage, chunk],
                dst_ref=recv_ref.at[stage, chunk],
                send_sem=send_sems.at[s],
                recv_sem=recv_sems.at[s],
                device_id=(partner[stage],),
                device_id_type=pl.DeviceIdType.MESH,
            )
            rdma.start()
            return rdma

        h = jnp.maximum(
            jnp.dot(x_ref[:, :], win0_ref[:, :], preferred_element_type=jnp.float32),
            0.0,
        )
        for layer in range(N_LAYERS):
            wout = wouts[layer]

            p = []
            r1 = []
            for c in range(N_CHUNKS):
                pc = jnp.dot(
                    h, wout[:, c * dc : (c + 1) * dc],
                    preferred_element_type=jnp.float32,
                )
                p.append(pc)
                r1.append(exchange(layer, 0, c, pc))

            acc = []
            r2 = []
            for c in range(N_CHUNKS):
                r1[c].wait()
                ac = p[c] + recv_ref[0, c]
                acc.append(ac)
                r2.append(exchange(layer, 1, c, ac))

            if layer == N_LAYERS - 1:
                for c in range(N_CHUNKS):
                    r2[c].wait()
                    out_ref[:, c * dc : (c + 1) * dc] = acc[c] + recv_ref[1, c]
            else:
                win = wins[layer + 1]
                t = None
                for c in range(N_CHUNKS):
                    r2[c].wait()
                    xc = acc[c] + recv_ref[1, c]
                    tc = jnp.dot(
                        xc, win[c * dc : (c + 1) * dc, :],
                        preferred_element_type=jnp.float32,
                    )
                    t = tc if t is None else t + tc
                h = jnp.maximum(t, 0.0)

    return pl.pallas_call(
        body,
        out_shape=jax.ShapeDtypeStruct((b, d), jnp.float32),
        in_specs=[pl.BlockSpec(memory_space=pltpu.VMEM)] * 7,
        out_specs=pl.BlockSpec(memory_space=pltpu.VMEM),
        scratch_shapes=[
            pltpu.VMEM((N_STAGES, N_CHUNKS, b, dc), jnp.float32),
            pltpu.VMEM((N_STAGES, N_CHUNKS, b, dc), jnp.float32),
            pltpu.SemaphoreType.DMA((N_LAYERS * N_STAGES * N_CHUNKS,)),
            pltpu.SemaphoreType.DMA((N_LAYERS * N_STAGES * N_CHUNKS,)),
        ],
        compiler_params=pltpu.CompilerParams(
            collective_id=0,
            vmem_limit_bytes=100 * 1024 * 1024,
        ),
    )(x, Win0, Wout0, Win1, Wout1, Win2, Wout2)
